# baseline (speedup 1.0000x reference)
import jax
import jax.numpy as jnp
from jax import lax
from jax.experimental import pallas as pl
from jax.experimental.pallas import tpu as pltpu, tpu_sc as plsc

NUM_USER = 25000
NUM_ITEM = 25000
N_NODES = NUM_USER + NUM_ITEM
EMBED_DIM = 64
N_EDGES = 800000
N_LAYERS = 4

NC, NS = 2, 16
NQ = 4
QROWS = N_NODES // NQ
ACC_ROWS = QROWS + 12

NV = 64
RT_E = 12512
E_PAD = NV * RT_E
CHUNK = 128
REGION = 13312
R_CHUNKS = REGION // CHUNK
RBUF = REGION + 16
TRASH = REGION
BLK_CHUNKS = R_CHUNKS // 2
BLK_E = BLK_CHUNKS * CHUNK
QPB = BLK_CHUNKS // 4
NBUF = 4

OUT_ROWS = N_NODES + 16
COPY_ROWS = 250
N_COPY = QROWS // COPY_ROWS


def _route_body(src_hbm, dst_hbm, val_hbm,
                src_c, dloc_c, val_c, counts,
                inb_s, inb_d, inb_v, osrc, odloc, oval, cntv):
    core = lax.axis_index("c")
    tile = lax.axis_index("s")
    wid = core * NS + tile

    for p in range(NV // (NC * NS)):
        vt = wid + NC * NS * p
        in_base = vt * RT_E
        pltpu.sync_copy(src_hbm.at[pl.ds(in_base, RT_E)], inb_s)
        pltpu.sync_copy(dst_hbm.at[pl.ds(in_base, RT_E)], inb_d)
        pltpu.sync_copy(val_hbm.at[pl.ds(in_base, RT_E)], inb_v)

        for m in range(2):
            zi = jnp.zeros((16,), jnp.int32)
            zf = jnp.zeros((16,), jnp.float32)

            def zero16(k, carry):
                sl = pl.ds(k * 16, 16)
                for j in range(2):
                    osrc[j][sl] = zi
                    odloc[j][sl] = zi
                    oval[j][sl] = zf
                return carry

            lax.fori_loop(0, RBUF // 16, zero16, 0)

            def group_body(g, ptrs):
                sl = pl.ds(g * 16, 16)
                s16 = inb_s[sl]
                d16 = inb_d[sl]
                v16 = inb_v[sl]
                lanes = lax.iota(jnp.int32, 16)
                new_ptrs = []
                for j in range(2):
                    q = 2 * m + j
                    dl16 = d16 - q * QROWS
                    mk = (dl16 >= 0) & (dl16 < QROWS)
                    mi = jnp.where(mk, jnp.ones((16,), jnp.int32),
                                   jnp.zeros((16,), jnp.int32))
                    v = mi
                    for k in (1, 2, 4, 8):
                        idx = jnp.maximum(lanes - k, 0)
                        sh = v.at[idx].get(mode="promise_in_bounds")
                        v = v + jnp.where(lanes >= k, sh,
                                          jnp.zeros((16,), jnp.int32))
                    cnt = v[15]
                    ptr = ptrs[j]
                    pos = jnp.where(mk, ptr + (v - mi), TRASH)
                    plsc.store_scatter(osrc[j], [pos], s16)
                    plsc.store_scatter(odloc[j], [pos], dl16)
                    plsc.store_scatter(oval[j], [pos], v16)
                    new_ptrs.append(ptr + cnt)
                return tuple(new_ptrs)

            ptrs = lax.fori_loop(0, RT_E // 16, group_body,
                                 (jnp.int32(0), jnp.int32(0)))

            for j in range(2):
                rid = (2 * m + j) * NV + vt
                region_off = rid * REGION
                pltpu.sync_copy(osrc[j].at[pl.ds(0, REGION)],
                                src_c.at[pl.ds(region_off, REGION)])
                pltpu.sync_copy(odloc[j].at[pl.ds(0, REGION)],
                                dloc_c.at[pl.ds(region_off, REGION)])
                pltpu.sync_copy(oval[j].at[pl.ds(0, REGION)],
                                val_c.at[pl.ds(region_off, REGION)])
                cntv[pl.ds(0, 16)] = jnp.zeros((16,), jnp.int32) + ptrs[j]
                pltpu.sync_copy(cntv, counts.at[pl.ds(rid * 16, 16)])


def _layer_body(embeds_hbm, src_c, dloc2d, val_c, counts, zeros_hbm, out_hbm,
                acc, srcb, dlocb, valb, rows, cntb, gs, ss, ib):
    core = lax.axis_index("c")
    tile = lax.axis_index("s")

    dummy = zeros_hbm.at[pl.ds(0, CHUNK)]

    def drain(sem):
        pltpu.make_async_copy(dummy, rows[0], sem).wait()

    def drain_blk():
        pltpu.make_async_copy(src_c.at[pl.ds(0, BLK_E)],
                              srcb.at[pl.ds(0, BLK_E)], ib).wait()
        pltpu.make_async_copy(val_c.at[pl.ds(0, BLK_E)],
                              valb.at[pl.ds(0, BLK_E)], ib).wait()
        pltpu.make_async_copy(dloc2d.at[pl.ds(0, BLK_CHUNKS)],
                              dlocb.at[pl.ds(0, BLK_CHUNKS)], ib).wait()

    def scale(rbuf, voff):
        for g in range(CHUNK // 16):
            v16 = valb[pl.ds(voff + g * 16, 16)]
            for e in range(16):
                ge = g * 16 + e
                v = v16[e]
                for j in range(EMBED_DIM // 16):
                    col = pl.ds(j * 16, 16)
                    rbuf[ge, col] = rbuf[ge, col] * v

    for ph in range(2):
        q = 2 * ph + core

        for k in range(4):
            cidx = tile + NS * k
            @pl.when(cidx < N_COPY)
            def _():
                pltpu.sync_copy(
                    zeros_hbm.at[pl.ds(cidx * COPY_ROWS, COPY_ROWS)],
                    acc.at[pl.ds(cidx * COPY_ROWS, COPY_ROWS)])
        plsc.subcore_barrier()

        def region_body(p, carry):
            rid = q * NV + tile + NS * p
            region_off = rid * REGION

            pltpu.sync_copy(counts.at[pl.ds(rid * 16, 16)], cntb)
            cnt = cntb[pl.ds(0, 16)][0]
            nquads = lax.div(cnt + NBUF * CHUNK - 1, NBUF * CHUNK)
            nblocks = lax.div(nquads + QPB - 1, QPB)

            def load_block(b, par):
                base = region_off + b * BLK_E
                dst = pl.ds(par * BLK_E, BLK_E)
                pltpu.async_copy(src_c.at[pl.ds(base, BLK_E)],
                                 srcb.at[dst], ib)
                pltpu.async_copy(val_c.at[pl.ds(base, BLK_E)],
                                 valb.at[dst], ib)
                pltpu.async_copy(
                    dloc2d.at[pl.ds(rid * R_CHUNKS + b * BLK_CHUNKS,
                                    BLK_CHUNKS)],
                    dlocb.at[pl.ds(par * BLK_CHUNKS, BLK_CHUNKS)], ib)

            @pl.when(nquads > 0)
            def _():
                load_block(0, 0)
                drain_blk()
                @pl.when(nblocks > 1)
                def _():
                    load_block(1, 1)
                for k in range(NBUF):
                    pltpu.async_copy(
                        embeds_hbm.at[srcb.at[pl.ds(k * CHUNK, CHUNK)]],
                        rows[k], gs[k])

                def quad_body(i, carry2):
                    b = lax.div(i, QPB)
                    lb = lax.rem(i, QPB)
                    par = lax.rem(b, 2)
                    base_off = par * BLK_E + lb * (NBUF * CHUNK)
                    row_base = par * BLK_CHUNKS + lb * NBUF

                    for k in range(NBUF):
                        drain(gs[k])
                        scale(rows[k], base_off + k * CHUNK)
                        pltpu.async_copy(rows[k],
                                         acc.at[dlocb.at[row_base + k]],
                                         ss[k], add=True)

                    nxt = i + 1
                    @pl.when(nxt < nquads)
                    def _():
                        nb = lax.div(nxt, QPB)
                        npar = lax.rem(nb, 2)
                        @pl.when(lax.rem(nxt, QPB) == 0)
                        def _():
                            drain_blk()
                        nbase = npar * BLK_E + lax.rem(nxt, QPB) * (NBUF *
                                                                    CHUNK)
                        for k in range(NBUF):
                            drain(ss[k])
                            pltpu.async_copy(
                                embeds_hbm.at[
                                    srcb.at[pl.ds(nbase + k * CHUNK, CHUNK)]],
                                rows[k], gs[k])
                        @pl.when(jnp.logical_and(lax.rem(nxt, QPB) == 0,
                                                 nb + 1 < nblocks))
                        def _():
                            load_block(nb + 1, 1 - npar)
                    return carry2

                lax.fori_loop(0, nquads, quad_body, 0)
                for k in range(NBUF):
                    drain(ss[k])
            return carry

        lax.fori_loop(0, NV // NS, region_body, 0)
        plsc.subcore_barrier()

        out_base = q * QROWS
        for k in range(4):
            cidx = tile + NS * k
            @pl.when(cidx < N_COPY)
            def _():
                pltpu.sync_copy(
                    acc.at[pl.ds(cidx * COPY_ROWS, COPY_ROWS)],
                    out_hbm.at[pl.ds(out_base + cidx * COPY_ROWS,
                                     COPY_ROWS)])
        plsc.subcore_barrier()


@jax.jit
def _route(src, dst, vals):
    mesh = plsc.VectorSubcoreMesh(core_axis_name="c", subcore_axis_name="s")
    n = NQ * NV * REGION
    f = pl.kernel(
        _route_body,
        out_type=(
            jax.ShapeDtypeStruct((n,), jnp.int32),
            jax.ShapeDtypeStruct((n,), jnp.int32),
            jax.ShapeDtypeStruct((n,), jnp.float32),
            jax.ShapeDtypeStruct((NQ * NV * 16,), jnp.int32),
        ),
        mesh=mesh,
        scratch_types=[
            pltpu.VMEM((RT_E,), jnp.int32),
            pltpu.VMEM((RT_E,), jnp.int32),
            pltpu.VMEM((RT_E,), jnp.float32),
            [pltpu.VMEM((RBUF,), jnp.int32) for _ in range(2)],
            [pltpu.VMEM((RBUF,), jnp.int32) for _ in range(2)],
            [pltpu.VMEM((RBUF,), jnp.float32) for _ in range(2)],
            pltpu.VMEM((16,), jnp.int32),
        ],
        compiler_params=pltpu.CompilerParams(use_tc_tiling_on_sc=False,
                                             needs_layout_passes=False),
    )
    return f(src, dst, vals)


@jax.jit
def _run_layer(embeds, src_c, dloc2d, val_c, counts, zeros):
    mesh = plsc.VectorSubcoreMesh(core_axis_name="c", subcore_axis_name="s")
    f = pl.kernel(
        _layer_body,
        out_type=jax.ShapeDtypeStruct((OUT_ROWS, EMBED_DIM), jnp.float32),
        mesh=mesh,
        scratch_types=[
            pltpu.VMEM_SHARED((ACC_ROWS, EMBED_DIM), jnp.float32),
            pltpu.VMEM((2 * BLK_E,), jnp.int32),
            pltpu.VMEM((2 * BLK_CHUNKS, CHUNK), jnp.int32),
            pltpu.VMEM((2 * BLK_E,), jnp.float32),
            [pltpu.VMEM((CHUNK, EMBED_DIM), jnp.float32)
             for _ in range(NBUF)],
            pltpu.VMEM((16,), jnp.int32),
            [pltpu.SemaphoreType.DMA for _ in range(NBUF)],
            [pltpu.SemaphoreType.DMA for _ in range(NBUF)],
            pltpu.SemaphoreType.DMA,
        ],
        compiler_params=pltpu.CompilerParams(use_tc_tiling_on_sc=False,
                                             needs_layout_passes=False),
    )
    return f(embeds, src_c, dloc2d, val_c, counts, zeros)


def kernel(user_emb, item_emb, edge_vals, edge_index):
    embeds = jnp.concatenate([user_emb, item_emb], axis=0)
    embeds = jnp.pad(embeds, ((0, OUT_ROWS - N_NODES), (0, 0)))
    pad_e = E_PAD - N_EDGES
    src = jnp.pad(edge_index[0], (0, pad_e))
    dst = jnp.pad(edge_index[1], (0, pad_e), constant_values=N_NODES + 1)
    vals = jnp.pad(edge_vals, (0, pad_e))
    zeros = jnp.zeros((QROWS, EMBED_DIM), jnp.float32)

    src_c, dloc_c, val_c, counts = _route(src, dst, vals)
    dloc2d = dloc_c.reshape(NQ * NV * R_CHUNKS, CHUNK)
    for _ in range(N_LAYERS):
        embeds = _run_layer(embeds, src_c, dloc2d, val_c, counts, zeros)
    return embeds[:NUM_USER], embeds[NUM_USER:N_NODES]


# speedup vs baseline: 3.8298x; 3.8298x over previous
import jax
import jax.numpy as jnp
from jax import lax
from jax.experimental import pallas as pl
from jax.experimental.pallas import tpu as pltpu, tpu_sc as plsc

NUM_USER = 25000
NUM_ITEM = 25000
N_NODES = NUM_USER + NUM_ITEM
EMBED_DIM = 64
N_EDGES = 800000
N_LAYERS = 4

NC, NS = 2, 16
HALF = N_NODES // NC
ACC_ROWS = HALF + 8

NV = 64
RT_E = 12512
E_PAD = NV * RT_E
CHUNK = 128
REGION = 12544
R_CHUNKS = REGION // CHUNK
RBUF = REGION + 16
TRASH = REGION
BLK_CHUNKS = 14
BLK_PAIRS = BLK_CHUNKS // 2
BLK_E = BLK_CHUNKS * CHUNK

OUT_ROWS = N_NODES + 16
COPY_ROWS = 200
N_COPY = HALF // COPY_ROWS


def _route_body(src_hbm, dst_hbm, val_hbm,
                src_c, dloc_c, val_c, counts,
                inb_s, inb_d, inb_v, osrc, odloc, oval, cntv):
    core = lax.axis_index("c")
    tile = lax.axis_index("s")
    wid = core * NS + tile

    for p in range(NV // (NC * NS)):
        vt = wid + NC * NS * p
        in_base = vt * RT_E
        pltpu.sync_copy(src_hbm.at[pl.ds(in_base, RT_E)], inb_s)
        pltpu.sync_copy(dst_hbm.at[pl.ds(in_base, RT_E)], inb_d)
        pltpu.sync_copy(val_hbm.at[pl.ds(in_base, RT_E)], inb_v)

        zi = jnp.zeros((16,), jnp.int32)
        zf = jnp.zeros((16,), jnp.float32)

        def zero16(k, carry):
            sl = pl.ds(k * 16, 16)
            for h in range(NC):
                osrc[h][sl] = zi
                odloc[h][sl] = zi
                oval[h][sl] = zf
            return carry

        lax.fori_loop(0, RBUF // 16, zero16, 0)

        def group_body(g, ptrs):
            sl = pl.ds(g * 16, 16)
            s16 = inb_s[sl]
            d16 = inb_d[sl]
            v16 = inb_v[sl]
            lanes = lax.iota(jnp.int32, 16)
            new_ptrs = []
            for h in range(NC):
                dl16 = d16 - h * HALF
                m = (dl16 >= 0) & (dl16 < HALF)
                mi = jnp.where(m, jnp.ones((16,), jnp.int32),
                               jnp.zeros((16,), jnp.int32))
                v = mi
                for k in (1, 2, 4, 8):
                    idx = jnp.maximum(lanes - k, 0)
                    sh = v.at[idx].get(mode="promise_in_bounds")
                    v = v + jnp.where(lanes >= k, sh,
                                      jnp.zeros((16,), jnp.int32))
                cnt = v[15]
                ptr = ptrs[h]
                pos = jnp.where(m, ptr + (v - mi), TRASH)
                plsc.store_scatter(osrc[h], [pos], s16)
                plsc.store_scatter(odloc[h], [pos], dl16)
                plsc.store_scatter(oval[h], [pos], v16)
                new_ptrs.append(ptr + cnt)
            return tuple(new_ptrs)

        ptrs = lax.fori_loop(0, RT_E // 16, group_body,
                             (jnp.int32(0), jnp.int32(0)))

        for h in range(NC):
            region_off = (h * NV + vt) * REGION
            pltpu.sync_copy(osrc[h].at[pl.ds(0, REGION)],
                            src_c.at[pl.ds(region_off, REGION)])
            pltpu.sync_copy(odloc[h].at[pl.ds(0, REGION)],
                            dloc_c.at[pl.ds(region_off, REGION)])
            pltpu.sync_copy(oval[h].at[pl.ds(0, REGION)],
                            val_c.at[pl.ds(region_off, REGION)])
            cntv[pl.ds(0, 16)] = jnp.zeros((16,), jnp.int32) + ptrs[h]
            pltpu.sync_copy(cntv, counts.at[pl.ds((h * NV + vt) * 16, 16)])


def _layer_body(embeds_hbm, src_c, dloc2d, val_c, counts, zeros_hbm, out_hbm,
                acc, srcb, dlocb, valb, rows0, rows1, cntb, g0, g1, s0, s1, ib):
    core = lax.axis_index("c")
    tile = lax.axis_index("s")
    out_base = core * HALF

    for k in range(8):
        cidx = tile + NS * k
        @pl.when(cidx < N_COPY)
        def _():
            pltpu.sync_copy(zeros_hbm.at[pl.ds(cidx * COPY_ROWS, COPY_ROWS)],
                            acc.at[pl.ds(cidx * COPY_ROWS, COPY_ROWS)])
    plsc.subcore_barrier()

    dummy = zeros_hbm.at[pl.ds(0, CHUNK)]
    dummy_b = zeros_hbm.at[pl.ds(0, 3 * BLK_E // 64)]

    def drain(sem):
        pltpu.make_async_copy(dummy, rows0, sem).wait()

    def drain_blk():
        pltpu.make_async_copy(dummy_b, rows0.at[pl.ds(0, 3 * BLK_E // 64)],
                              ib).wait()

    def scale(rows, voff):
        for g in range(CHUNK // 16):
            v16 = valb[pl.ds(voff + g * 16, 16)]
            for e in range(16):
                ge = g * 16 + e
                v = v16[e]
                for j in range(EMBED_DIM // 16):
                    col = pl.ds(j * 16, 16)
                    rows[ge, col] = rows[ge, col] * v

    for p in range(4):
        vt = tile + NS * p
        rid = core * NV + vt
        region_off = rid * REGION

        pltpu.sync_copy(counts.at[pl.ds(rid * 16, 16)], cntb)
        cnt = cntb[pl.ds(0, 16)][0]
        npairs = lax.div(cnt + 2 * CHUNK - 1, 2 * CHUNK)
        nblocks = lax.div(npairs + BLK_PAIRS - 1, BLK_PAIRS)

        def load_block(b, par):
            base = region_off + b * BLK_E
            dst = pl.ds(par * BLK_E, BLK_E)
            pltpu.async_copy(src_c.at[pl.ds(base, BLK_E)], srcb.at[dst], ib)
            pltpu.async_copy(val_c.at[pl.ds(base, BLK_E)], valb.at[dst], ib)
            pltpu.async_copy(
                dloc2d.at[pl.ds(rid * R_CHUNKS + b * BLK_CHUNKS, BLK_CHUNKS)],
                dlocb.at[pl.ds(par * BLK_CHUNKS, BLK_CHUNKS)], ib)

        @pl.when(npairs > 0)
        def _():
            load_block(0, 0)
            drain_blk()
            pltpu.async_copy(embeds_hbm.at[srcb.at[pl.ds(0, CHUNK)]],
                             rows0, g0)

            def pair_body(i, carry):
                b = lax.div(i, BLK_PAIRS)
                lp = lax.rem(i, BLK_PAIRS)
                par = lax.rem(b, 2)
                o0 = par * BLK_E + lp * 2 * CHUNK
                o1 = o0 + CHUNK
                row0 = par * BLK_CHUNKS + lp * 2

                @pl.when(i > 0)
                def _():
                    drain(s1)

                @pl.when(jnp.logical_and(lp == 0, i > 0))
                def _():
                    drain_blk()
                @pl.when(jnp.logical_and(lp == 0, b + 1 < nblocks))
                def _():
                    load_block(b + 1, 1 - par)
                pltpu.async_copy(embeds_hbm.at[srcb.at[pl.ds(o1, CHUNK)]],
                                 rows1, g1)
                drain(g0)
                scale(rows0, o0)
                pltpu.async_copy(rows0, acc.at[dlocb.at[row0]], s0, add=True)
                drain(g1)
                drain(s0)

                @pl.when(i < npairs - 1)
                def _():
                    nb = lax.div(i + 1, BLK_PAIRS)
                    npar = lax.rem(nb, 2)
                    no0 = npar * BLK_E + lax.rem(i + 1, BLK_PAIRS) * 2 * CHUNK
                    pltpu.async_copy(embeds_hbm.at[srcb.at[pl.ds(no0, CHUNK)]],
                                     rows0, g0)
                scale(rows1, o1)
                pltpu.async_copy(rows1, acc.at[dlocb.at[row0 + 1]], s1,
                                 add=True)
                return carry

            lax.fori_loop(0, npairs, pair_body, 0)
            drain(s1)

    plsc.subcore_barrier()

    for k in range(8):
        cidx = tile + NS * k
        @pl.when(cidx < N_COPY)
        def _():
            pltpu.sync_copy(
                acc.at[pl.ds(cidx * COPY_ROWS, COPY_ROWS)],
                out_hbm.at[pl.ds(out_base + cidx * COPY_ROWS, COPY_ROWS)])


@jax.jit
def _route(src, dst, vals):
    mesh = plsc.VectorSubcoreMesh(core_axis_name="c", subcore_axis_name="s")
    n = NC * NV * REGION
    f = pl.kernel(
        _route_body,
        out_type=(
            jax.ShapeDtypeStruct((n,), jnp.int32),
            jax.ShapeDtypeStruct((n,), jnp.int32),
            jax.ShapeDtypeStruct((n,), jnp.float32),
            jax.ShapeDtypeStruct((NC * NV * 16,), jnp.int32),
        ),
        mesh=mesh,
        scratch_types=[
            pltpu.VMEM((RT_E,), jnp.int32),
            pltpu.VMEM((RT_E,), jnp.int32),
            pltpu.VMEM((RT_E,), jnp.float32),
            [pltpu.VMEM((RBUF,), jnp.int32) for _ in range(NC)],
            [pltpu.VMEM((RBUF,), jnp.int32) for _ in range(NC)],
            [pltpu.VMEM((RBUF,), jnp.float32) for _ in range(NC)],
            pltpu.VMEM((16,), jnp.int32),
        ],
        compiler_params=pltpu.CompilerParams(use_tc_tiling_on_sc=False,
                                             needs_layout_passes=False),
    )
    return f(src, dst, vals)


@jax.jit
def _run_layer(embeds, src_c, dloc2d, val_c, counts, zeros):
    mesh = plsc.VectorSubcoreMesh(core_axis_name="c", subcore_axis_name="s")
    f = pl.kernel(
        _layer_body,
        out_type=jax.ShapeDtypeStruct((OUT_ROWS, EMBED_DIM), jnp.float32),
        mesh=mesh,
        scratch_types=[
            pltpu.VMEM_SHARED((ACC_ROWS, EMBED_DIM), jnp.float32),
            pltpu.VMEM((2 * BLK_E,), jnp.int32),
            pltpu.VMEM((2 * BLK_CHUNKS, CHUNK), jnp.int32),
            pltpu.VMEM((2 * BLK_E,), jnp.float32),
            pltpu.VMEM((CHUNK, EMBED_DIM), jnp.float32),
            pltpu.VMEM((CHUNK, EMBED_DIM), jnp.float32),
            pltpu.VMEM((16,), jnp.int32),
            pltpu.SemaphoreType.DMA,
            pltpu.SemaphoreType.DMA,
            pltpu.SemaphoreType.DMA,
            pltpu.SemaphoreType.DMA,
            pltpu.SemaphoreType.DMA,
        ],
        compiler_params=pltpu.CompilerParams(use_tc_tiling_on_sc=False,
                                             needs_layout_passes=False),
    )
    return f(embeds, src_c, dloc2d, val_c, counts, zeros)


def kernel(user_emb, item_emb, edge_vals, edge_index):
    embeds = jnp.concatenate([user_emb, item_emb], axis=0)
    embeds = jnp.pad(embeds, ((0, OUT_ROWS - N_NODES), (0, 0)))
    pad_e = E_PAD - N_EDGES
    src = jnp.pad(edge_index[0], (0, pad_e))
    dst = jnp.pad(edge_index[1], (0, pad_e), constant_values=N_NODES + 1)
    vals = jnp.pad(edge_vals, (0, pad_e))
    zeros = jnp.zeros((HALF, EMBED_DIM), jnp.float32)

    src_c, dloc_c, val_c, counts = _route(src, dst, vals)
    dloc2d = dloc_c.reshape(NC * NV * R_CHUNKS, CHUNK)
    for _ in range(N_LAYERS):
        embeds = _run_layer(embeds, src_c, dloc2d, val_c, counts, zeros)
    return embeds[:NUM_USER], embeds[NUM_USER:N_NODES]
